# baseline (device time: 15350 ns/iter reference)
import jax
import jax.numpy as jnp
from jax import lax
from jax.experimental import pallas as pl
from jax.experimental.pallas import tpu as pltpu

N_DEV = 4


def kernel(x, router_W, route_idx, expert_W):
    n, d = x.shape
    e_per, _, h = expert_W.shape

    def body(x_ref, rw_ref, idx_ref, ew_ref, out_ref,
             comm_ref, send_sems, recv_sems):
        my_i = lax.axis_index("i")
        left = (my_i + N_DEV - 1) % N_DEV
        right = (my_i + 1) % N_DEV

        barrier_sem = pltpu.get_barrier_semaphore()
        for nbr in (left, right):
            pl.semaphore_signal(
                barrier_sem, inc=1,
                device_id=(nbr,), device_id_type=pl.DeviceIdType.MESH,
            )
        pl.semaphore_wait(barrier_sem, 2)

        xb = x_ref[:, :].astype(jnp.bfloat16)
        idx = idx_ref[:, :]
        acc = jnp.zeros((n, h), jnp.float32)
        for k in range(e_per):
            e_glob = my_i * e_per + k
            xm = jnp.where(idx == e_glob, xb, jnp.bfloat16(0.0))
            w = ew_ref[k, :, :].astype(jnp.bfloat16)
            acc = acc + jnp.dot(xm, w, preferred_element_type=jnp.float32)

        out_ref[:, :] = acc
        comm_ref[0, :, :] = acc.astype(jnp.bfloat16)

        for t in range(N_DEV - 1):
            rdma = pltpu.make_async_remote_copy(
                src_ref=comm_ref.at[t],
                dst_ref=comm_ref.at[t + 1],
                send_sem=send_sems.at[t],
                recv_sem=recv_sems.at[t],
                device_id=(right,),
                device_id_type=pl.DeviceIdType.MESH,
            )
            rdma.start()
            rdma.wait()
            out_ref[:, :] += comm_ref[t + 1, :, :].astype(jnp.float32)

    return pl.pallas_call(
        body,
        out_shape=jax.ShapeDtypeStruct((n, h), jnp.float32),
        in_specs=[pl.BlockSpec(memory_space=pltpu.VMEM)] * 4,
        out_specs=pl.BlockSpec(memory_space=pltpu.VMEM),
        scratch_shapes=[
            pltpu.VMEM((N_DEV, n, h), jnp.bfloat16),
            pltpu.SemaphoreType.DMA((N_DEV - 1,)),
            pltpu.SemaphoreType.DMA((N_DEV - 1,)),
        ],
        compiler_params=pltpu.CompilerParams(collective_id=0),
    )(x, router_W, route_idx, expert_W)


# device time: 12414 ns/iter; 1.2365x vs baseline; 1.2365x over previous
import functools

import jax
import jax.numpy as jnp
from jax import lax
from jax.experimental import pallas as pl
from jax.experimental.pallas import tpu as pltpu

N_DEV = 4


def kernel(x, router_W, route_idx, expert_W):
    n, d = x.shape
    e_per, _, h = expert_W.shape

    def body(x_ref, rw_ref, idx_ref, ew_ref, out_ref,
             comm_ref, send_sems, recv_sems):
        my_i = lax.axis_index("i")
        left = (my_i + N_DEV - 1) % N_DEV
        right = (my_i + 1) % N_DEV
        diag = (my_i + 2) % N_DEV
        peers = (right, left, diag)

        barrier_sem = pltpu.get_barrier_semaphore()
        for nbr in peers:
            pl.semaphore_signal(
                barrier_sem, inc=1,
                device_id=(nbr,), device_id_type=pl.DeviceIdType.MESH,
            )

        xb = x_ref[:, :].astype(jnp.bfloat16)
        idx = idx_ref[:, :]
        acc = jnp.zeros((n, h), jnp.float32)
        for k in range(e_per):
            e_glob = my_i * e_per + k
            xm = jnp.where(idx == e_glob, xb, jnp.bfloat16(0.0))
            w = ew_ref[k, :, :].astype(jnp.bfloat16)
            acc = acc + jnp.dot(xm, w, preferred_element_type=jnp.float32)
        comm_ref[0, :, :] = acc.astype(jnp.bfloat16)

        pl.semaphore_wait(barrier_sem, 3)

        rdmas = []
        for s, tgt in ((1, right), (2, left), (3, diag)):
            rdma = pltpu.make_async_remote_copy(
                src_ref=comm_ref.at[0],
                dst_ref=comm_ref.at[s],
                send_sem=send_sems.at[s - 1],
                recv_sem=recv_sems.at[s - 1],
                device_id=(tgt,),
                device_id_type=pl.DeviceIdType.MESH,
            )
            rdma.start()
            rdmas.append(rdma)

        out_ref[:, :] = acc
        for s, rdma in zip((1, 2, 3), rdmas):
            rdma.wait()
            out_ref[:, :] += comm_ref[s, :, :].astype(jnp.float32)

        @functools.partial(
            pl.run_scoped, exit_sem=pltpu.SemaphoreType.REGULAR
        )
        def _(exit_sem):
            for nbr in peers:
                pl.semaphore_signal(
                    exit_sem, inc=1,
                    device_id=(nbr,), device_id_type=pl.DeviceIdType.MESH,
                )
            pl.semaphore_wait(exit_sem, 3)

    return pl.pallas_call(
        body,
        out_shape=jax.ShapeDtypeStruct((n, h), jnp.float32),
        in_specs=[pl.BlockSpec(memory_space=pltpu.VMEM)] * 4,
        out_specs=pl.BlockSpec(memory_space=pltpu.VMEM),
        scratch_shapes=[
            pltpu.VMEM((N_DEV, n, h), jnp.bfloat16),
            pltpu.SemaphoreType.DMA((3,)),
            pltpu.SemaphoreType.DMA((3,)),
        ],
        compiler_params=pltpu.CompilerParams(collective_id=0),
    )(x, router_W, route_idx, expert_W)


# device time: 10366 ns/iter; 1.4808x vs baseline; 1.1976x over previous
import jax
import jax.numpy as jnp
from jax import lax
from jax.experimental import pallas as pl
from jax.experimental.pallas import tpu as pltpu

N_DEV = 4


def kernel(x, router_W, route_idx, expert_W):
    n, d = x.shape
    e_per, _, h = expert_W.shape

    def body(x_ref, rw_ref, idx_ref, ew_ref, out_ref,
             comm_ref, send_sems, recv_sems):
        my_i = lax.axis_index("i")
        left = (my_i + N_DEV - 1) % N_DEV
        right = (my_i + 1) % N_DEV
        diag = (my_i + 2) % N_DEV

        barrier_sem = pltpu.get_barrier_semaphore()
        for nbr in (right, left, diag):
            pl.semaphore_signal(
                barrier_sem, inc=1,
                device_id=(nbr,), device_id_type=pl.DeviceIdType.MESH,
            )

        xb = x_ref[:, :].astype(jnp.bfloat16)
        idx = idx_ref[:, :]
        e0 = my_i * e_per
        xm = jnp.concatenate(
            [jnp.where(idx == e0 + k, xb, jnp.bfloat16(0.0))
             for k in range(e_per)],
            axis=1,
        )
        ws = ew_ref[:, :, :].astype(jnp.bfloat16).reshape(e_per * d, h)
        acc = jnp.dot(xm, ws, preferred_element_type=jnp.float32)
        comm_ref[0, :, :] = acc.astype(jnp.bfloat16)

        pl.semaphore_wait(barrier_sem, 3)

        rdmas = []
        for s, tgt in ((1, right), (2, left), (3, diag)):
            rdma = pltpu.make_async_remote_copy(
                src_ref=comm_ref.at[0],
                dst_ref=comm_ref.at[s],
                send_sem=send_sems.at[s - 1],
                recv_sem=recv_sems.at[s - 1],
                device_id=(tgt,),
                device_id_type=pl.DeviceIdType.MESH,
            )
            rdma.start()
            rdmas.append(rdma)
        out_ref[:, :] = acc

        rdmas[0].wait_recv()
        rdmas[1].wait_recv()
        out_ref[:, :] += (comm_ref[1, :, :] + comm_ref[2, :, :]).astype(
            jnp.float32
        )
        rdmas[2].wait_recv()
        out_ref[:, :] += comm_ref[3, :, :].astype(jnp.float32)

        for rdma in rdmas:
            rdma.wait_send()

    return pl.pallas_call(
        body,
        out_shape=jax.ShapeDtypeStruct((n, h), jnp.float32),
        in_specs=[pl.BlockSpec(memory_space=pltpu.VMEM)] * 4,
        out_specs=pl.BlockSpec(memory_space=pltpu.VMEM),
        scratch_shapes=[
            pltpu.VMEM((N_DEV, n, h), jnp.bfloat16),
            pltpu.SemaphoreType.DMA((3,)),
            pltpu.SemaphoreType.DMA((3,)),
        ],
        compiler_params=pltpu.CompilerParams(collective_id=0),
    )(x, router_W, route_idx, expert_W)
